# baseline (speedup 1.0000x reference)
import numpy as np
import jax
import jax.numpy as jnp
from jax.experimental import pallas as pl

_P = 3
_Q = 3
_NCU = 32
_NCV = 32
_OUT_U = 128
_OUT_V = 128
_NB = 4


def _knots(n_ctrl, p):
    interior = np.arange(1, n_ctrl - p, dtype=np.float64) / float(n_ctrl - p)
    return np.concatenate([np.zeros(p + 1), interior, np.ones(p + 1)])


def _spans(n, p, u, U):
    eps = 1e-6
    out = np.zeros(len(u), dtype=np.int64)
    for i in range(len(u)):
        if abs(u[i] - U[n + 1]) < eps:
            out[i] = n
        else:
            low, high = p, n + 1
            mid = (low + high) // 2
            while u[i] < U[mid] - eps or u[i] >= U[mid + 1] + eps:
                if u[i] < U[mid] - eps:
                    high = mid
                else:
                    low = mid
                mid = (low + high) // 2
            out[i] = mid
    return out


def _basis_row(span, u, p, U):
    N = np.zeros(p + 1)
    left = np.zeros(p + 1)
    right = np.zeros(p + 1)
    N[0] = 1.0
    for j in range(1, p + 1):
        left[j] = u - U[span + 1 - j]
        right[j] = U[span + j] - u
        saved = 0.0
        for r in range(j):
            temp = N[r] / (right[r + 1] + left[j - r] + 1e-10)
            N[r] = saved + right[r + 1] * temp
            saved = left[j - r] * temp
        N[j] = saved
    return N


def _dense_basis(n_ctrl, p, n_out):
    U = _knots(n_ctrl, p)
    m = len(U) - p - 2
    samp = np.linspace(0.0, 1.0, n_out)
    spans = _spans(m, p, samp, U)
    A = np.zeros((n_out, n_ctrl), dtype=np.float32)
    for i in range(n_out):
        w = _basis_row(int(spans[i]), float(samp[i]), p, U)
        A[i, spans[i] - p : spans[i] + 1] = w.astype(np.float32)
    return A


_AU = _dense_basis(_NCU, _P, _OUT_U)
_AV = _dense_basis(_NCV, _Q, _OUT_V)

_MCAT4 = np.zeros((_NCV * 4, _OUT_V * 4), dtype=np.float32)
for _c in range(_NCV):
    for _d in range(4):
        _MCAT4[_c * 4 + _d, _d * _OUT_V : (_d + 1) * _OUT_V] = _AV[:, _c]


def _surf_body(x_ref, au_ref, mcat_ref, o_ref):
    au = au_ref[...]
    xall = x_ref[...].reshape(_NB * _NCU, _NCV * 4)
    y = jnp.dot(xall, mcat_ref[...], preferred_element_type=jnp.float32)
    for i in range(_NB):
        p = jnp.dot(au, y[i * _NCU : (i + 1) * _NCU], preferred_element_type=jnp.float32)
        r = 1.0 / jnp.maximum(p[:, 3 * _OUT_V :], 1e-8)
        o_ref[i, 0] = p[:, 0 * _OUT_V : 1 * _OUT_V] * r
        o_ref[i, 1] = p[:, 1 * _OUT_V : 2 * _OUT_V] * r
        o_ref[i, 2] = p[:, 2 * _OUT_V : 3 * _OUT_V] * r


def kernel(ctrl_pts):
    B = ctrl_pts.shape[0]
    x = ctrl_pts.reshape(B, _NCU, _NCV * 4)
    out = pl.pallas_call(
        _surf_body,
        grid=(B // _NB,),
        in_specs=[
            pl.BlockSpec((_NB, _NCU, _NCV * 4), lambda b: (b, 0, 0)),
            pl.BlockSpec((_OUT_U, _NCU), lambda b: (0, 0)),
            pl.BlockSpec((_NCV * 4, _OUT_V * 4), lambda b: (0, 0)),
        ],
        out_specs=pl.BlockSpec((_NB, 3, _OUT_U, _OUT_V), lambda b: (b, 0, 0, 0)),
        out_shape=jax.ShapeDtypeStruct((B, 3, _OUT_U, _OUT_V), jnp.float32),
    )(x, jnp.asarray(_AU), jnp.asarray(_MCAT4))
    return jnp.transpose(out, (0, 2, 3, 1))


# speedup vs baseline: 1.8249x; 1.8249x over previous
import numpy as np
import jax
import jax.numpy as jnp
from jax.experimental import pallas as pl

_P = 3
_Q = 3
_NCU = 32
_NCV = 32
_OUT_U = 128
_OUT_V = 128
_NB = 16


def _knots(n_ctrl, p):
    interior = np.arange(1, n_ctrl - p, dtype=np.float64) / float(n_ctrl - p)
    return np.concatenate([np.zeros(p + 1), interior, np.ones(p + 1)])


def _spans(n, p, u, U):
    eps = 1e-6
    out = np.zeros(len(u), dtype=np.int64)
    for i in range(len(u)):
        if abs(u[i] - U[n + 1]) < eps:
            out[i] = n
        else:
            low, high = p, n + 1
            mid = (low + high) // 2
            while u[i] < U[mid] - eps or u[i] >= U[mid + 1] + eps:
                if u[i] < U[mid] - eps:
                    high = mid
                else:
                    low = mid
                mid = (low + high) // 2
            out[i] = mid
    return out


def _basis_row(span, u, p, U):
    N = np.zeros(p + 1)
    left = np.zeros(p + 1)
    right = np.zeros(p + 1)
    N[0] = 1.0
    for j in range(1, p + 1):
        left[j] = u - U[span + 1 - j]
        right[j] = U[span + j] - u
        saved = 0.0
        for r in range(j):
            temp = N[r] / (right[r + 1] + left[j - r] + 1e-10)
            N[r] = saved + right[r + 1] * temp
            saved = left[j - r] * temp
        N[j] = saved
    return N


def _dense_basis(n_ctrl, p, n_out):
    U = _knots(n_ctrl, p)
    m = len(U) - p - 2
    samp = np.linspace(0.0, 1.0, n_out)
    spans = _spans(m, p, samp, U)
    A = np.zeros((n_out, n_ctrl), dtype=np.float32)
    for i in range(n_out):
        w = _basis_row(int(spans[i]), float(samp[i]), p, U)
        A[i, spans[i] - p : spans[i] + 1] = w.astype(np.float32)
    return A


_AU = _dense_basis(_NCU, _P, _OUT_U)
_AV = _dense_basis(_NCV, _Q, _OUT_V)

_MCAT4 = np.zeros((_NCV * 4, _OUT_V * 4), dtype=np.float32)
for _c in range(_NCV):
    for _d in range(4):
        _MCAT4[_c * 4 + _d, _d * _OUT_V : (_d + 1) * _OUT_V] = _AV[:, _c]


def _surf_body(x_ref, au_ref, mcat_ref, o_ref):
    au = au_ref[...]
    xall = x_ref[...].reshape(_NB * _NCU, _NCV * 4)
    y = jnp.dot(xall, mcat_ref[...], preferred_element_type=jnp.float32)
    for i in range(_NB):
        p = jnp.dot(au, y[i * _NCU : (i + 1) * _NCU], preferred_element_type=jnp.float32)
        r = 1.0 / jnp.maximum(p[:, 3 * _OUT_V :], 1e-8)
        o_ref[i, 0] = p[:, 0 * _OUT_V : 1 * _OUT_V] * r
        o_ref[i, 1] = p[:, 1 * _OUT_V : 2 * _OUT_V] * r
        o_ref[i, 2] = p[:, 2 * _OUT_V : 3 * _OUT_V] * r


def kernel(ctrl_pts):
    B = ctrl_pts.shape[0]
    x = ctrl_pts.reshape(B, _NCU, _NCV * 4)
    out = pl.pallas_call(
        _surf_body,
        grid=(B // _NB,),
        in_specs=[
            pl.BlockSpec((_NB, _NCU, _NCV * 4), lambda b: (b, 0, 0)),
            pl.BlockSpec((_OUT_U, _NCU), lambda b: (0, 0)),
            pl.BlockSpec((_NCV * 4, _OUT_V * 4), lambda b: (0, 0)),
        ],
        out_specs=pl.BlockSpec((_NB, 3, _OUT_U, _OUT_V), lambda b: (b, 0, 0, 0)),
        out_shape=jax.ShapeDtypeStruct((B, 3, _OUT_U, _OUT_V), jnp.float32),
    )(x, jnp.asarray(_AU), jnp.asarray(_MCAT4))
    return jnp.transpose(out, (0, 2, 3, 1))


# speedup vs baseline: 1.9562x; 1.0719x over previous
import numpy as np
import jax
import jax.numpy as jnp
from jax.experimental import pallas as pl

_P = 3
_Q = 3
_NCU = 32
_NCV = 32
_OUT_U = 128
_OUT_V = 128
_NB = 32


def _knots(n_ctrl, p):
    interior = np.arange(1, n_ctrl - p, dtype=np.float64) / float(n_ctrl - p)
    return np.concatenate([np.zeros(p + 1), interior, np.ones(p + 1)])


def _spans(n, p, u, U):
    eps = 1e-6
    out = np.zeros(len(u), dtype=np.int64)
    for i in range(len(u)):
        if abs(u[i] - U[n + 1]) < eps:
            out[i] = n
        else:
            low, high = p, n + 1
            mid = (low + high) // 2
            while u[i] < U[mid] - eps or u[i] >= U[mid + 1] + eps:
                if u[i] < U[mid] - eps:
                    high = mid
                else:
                    low = mid
                mid = (low + high) // 2
            out[i] = mid
    return out


def _basis_row(span, u, p, U):
    N = np.zeros(p + 1)
    left = np.zeros(p + 1)
    right = np.zeros(p + 1)
    N[0] = 1.0
    for j in range(1, p + 1):
        left[j] = u - U[span + 1 - j]
        right[j] = U[span + j] - u
        saved = 0.0
        for r in range(j):
            temp = N[r] / (right[r + 1] + left[j - r] + 1e-10)
            N[r] = saved + right[r + 1] * temp
            saved = left[j - r] * temp
        N[j] = saved
    return N


def _dense_basis(n_ctrl, p, n_out):
    U = _knots(n_ctrl, p)
    m = len(U) - p - 2
    samp = np.linspace(0.0, 1.0, n_out)
    spans = _spans(m, p, samp, U)
    A = np.zeros((n_out, n_ctrl), dtype=np.float32)
    for i in range(n_out):
        w = _basis_row(int(spans[i]), float(samp[i]), p, U)
        A[i, spans[i] - p : spans[i] + 1] = w.astype(np.float32)
    return A


_AU = _dense_basis(_NCU, _P, _OUT_U)
_AV = _dense_basis(_NCV, _Q, _OUT_V)

_MCAT4 = np.zeros((_NCV * 4, _OUT_V * 4), dtype=np.float32)
for _c in range(_NCV):
    for _d in range(4):
        _MCAT4[_c * 4 + _d, _d * _OUT_V : (_d + 1) * _OUT_V] = _AV[:, _c]


def _surf_body(x_ref, au_ref, mcat_ref, o_ref):
    au = au_ref[...]
    xall = x_ref[...].reshape(_NB * _NCU, _NCV * 4)
    y = jnp.dot(xall, mcat_ref[...], preferred_element_type=jnp.float32)
    for i in range(_NB):
        p = jnp.dot(au, y[i * _NCU : (i + 1) * _NCU], preferred_element_type=jnp.float32)
        r = 1.0 / jnp.maximum(p[:, 3 * _OUT_V :], 1e-8)
        o_ref[i, 0] = p[:, 0 * _OUT_V : 1 * _OUT_V] * r
        o_ref[i, 1] = p[:, 1 * _OUT_V : 2 * _OUT_V] * r
        o_ref[i, 2] = p[:, 2 * _OUT_V : 3 * _OUT_V] * r


def kernel(ctrl_pts):
    B = ctrl_pts.shape[0]
    x = ctrl_pts.reshape(B, _NCU, _NCV * 4)
    out = pl.pallas_call(
        _surf_body,
        grid=(B // _NB,),
        in_specs=[
            pl.BlockSpec((_NB, _NCU, _NCV * 4), lambda b: (b, 0, 0)),
            pl.BlockSpec((_OUT_U, _NCU), lambda b: (0, 0)),
            pl.BlockSpec((_NCV * 4, _OUT_V * 4), lambda b: (0, 0)),
        ],
        out_specs=pl.BlockSpec((_NB, 3, _OUT_U, _OUT_V), lambda b: (b, 0, 0, 0)),
        out_shape=jax.ShapeDtypeStruct((B, 3, _OUT_U, _OUT_V), jnp.float32),
    )(x, jnp.asarray(_AU), jnp.asarray(_MCAT4))
    return jnp.transpose(out, (0, 2, 3, 1))
